# baseline (speedup 1.0000x reference)
import functools

import jax
import jax.numpy as jnp
from jax import lax
from jax.experimental import pallas as pl
from jax.experimental.pallas import tpu as pltpu
from jax.experimental.pallas import tpu_sc as plsc

N = 10000
E = 320000
D = 128
P = 16384

NC = 2
NS = 16
NW = NC * NS
L = 16

EW = E // NW
CB = 125
CS = 128
CH = EW // CB
ACC_ROWS = 10240
HR = ACC_ROWS // 128
DUMMY = N
NB = 2

EPW = 10240
PCB = 64
PCH = EPW // PCB
PPH = PCH // 2
PAD_PK = 10128 << 14

_mesh = plsc.VectorSubcoreMesh(
    core_axis_name="c", subcore_axis_name="s", num_cores=NC, num_subcores=NS)
_sc_params = pltpu.CompilerParams(needs_layout_passes=False)


def _seg_body(table_h, pidx_h, part_h,
              pidx_v, r0, r1, si0, si1, di0, di1, acc_s, g0, g1, s0, s1):
    c = lax.axis_index("c")
    s = lax.axis_index("s")
    wid = s * NC + c
    rows = (r0, r1)
    sib = (si0, si1)
    dib = (di0, di1)
    gsem = (g0, g1)
    ssem = (s0, s1)
    zeros16 = jnp.zeros((L,), jnp.float32)

    iota2 = lax.iota(jnp.int32, L) * 2

    def unpack(j, b):
        for k in range(4):
            pk = pidx_v[j, pl.ds(k * L, L)]
            s16 = lax.bitwise_and(pk, 0x3FFF)
            d16 = lax.shift_right_logical(pk, 14)
            p16 = lax.bitwise_and(s16, 1)
            even = lax.bitwise_and(s16, 0x3FFE)
            odd = lax.bitwise_or(s16, 1)
            junk = DUMMY + lax.bitwise_and(s16, 127)
            deven = jnp.where(p16 == 0, d16, junk)
            dodd = jnp.where(p16 == 1, d16, junk)
            pos = iota2 + (k * 2 * L)
            plsc.store_scatter(sib[b], [pos], even)
            plsc.store_scatter(sib[b], [pos + 1], odd)
            plsc.store_scatter(dib[b], [pos], deven)
            plsc.store_scatter(dib[b], [pos + 1], dodd)

    def zrow(i, _):
        for k in range(8):
            r0[i, pl.ds(k * L, L)] = zeros16
        return 0
    lax.fori_loop(0, CS, zrow, 0)
    for j in range(5):
        pltpu.sync_copy(r0, acc_s.at[pl.ds(s * 640 + j * CS, CS)])
    plsc.subcore_barrier()

    for p in range(2):
        pltpu.sync_copy(pidx_h.at[wid, pl.ds(p * PPH, PPH)], pidx_v)
        for b in range(NB):
            unpack(jnp.int32(b), b)
            pltpu.async_copy(table_h.at[sib[b]], rows[b], gsem[b])

        def group(g, _):
            base = g * NB
            for b in range(NB):
                pltpu.make_async_copy(table_h.at[sib[b]], rows[b], gsem[b]).wait()
                pltpu.async_copy(rows[b], acc_s.at[dib[b]], ssem[b], add=True)
            for b in range(NB):
                j = base + b
                pltpu.make_async_copy(rows[b], acc_s.at[dib[b]], ssem[b]).wait()
                jn = j + NB

                @pl.when(jn < PPH)
                def _():
                    unpack(jn, b)
                    pltpu.async_copy(table_h.at[sib[b]], rows[b], gsem[b])
            return 0
        lax.fori_loop(0, PPH // NB, group, 0)
    plsc.subcore_barrier()

    pltpu.sync_copy(acc_s.at[pl.ds(s * 640, 640)], part_h.at[c, pl.ds(s * 640, 640)])


_segsum = pl.kernel(
    _seg_body,
    out_type=jax.ShapeDtypeStruct((NC, ACC_ROWS, D), jnp.float32),
    mesh=_mesh,
    scratch_types=[
        pltpu.VMEM((PPH, PCB), jnp.int32),
        pltpu.VMEM((CS, D), jnp.float32),
        pltpu.VMEM((CS, D), jnp.float32),
        pltpu.VMEM((CS,), jnp.int32),
        pltpu.VMEM((CS,), jnp.int32),
        pltpu.VMEM((CS,), jnp.int32),
        pltpu.VMEM((CS,), jnp.int32),
        pltpu.VMEM_SHARED((ACC_ROWS, D), jnp.float32),
        pltpu.SemaphoreType.DMA,
        pltpu.SemaphoreType.DMA,
        pltpu.SemaphoreType.DMA,
        pltpu.SemaphoreType.DMA,
    ],
    compiler_params=_sc_params,
    name="sc_segsum")


def _hist_body(didx_h, cnt_h, didx_v, hist_v):
    c = lax.axis_index("c")
    s = lax.axis_index("s")
    wid = s * NC + c
    zeros16 = jnp.zeros((L,), jnp.float32)
    ones16 = jnp.ones((L,), jnp.float32)

    pltpu.sync_copy(didx_h.at[wid], didx_v)

    def zh(i, _):
        hist_v[pl.ds(i * L, L)] = zeros16
        return 0
    lax.fori_loop(0, ACC_ROWS // L, zh, 0)

    def step(i, _):
        di = didx_v[pl.ds(i * L, L)]
        plsc.addupdate_scatter(hist_v, [di], ones16)
        return 0
    lax.fori_loop(0, (CH * CS) // L, step, 0)

    pltpu.sync_copy(hist_v, cnt_h.at[pl.ds(wid * ACC_ROWS, ACC_ROWS)])


_hist = pl.kernel(
    _hist_body,
    out_type=jax.ShapeDtypeStruct((NW * ACC_ROWS,), jnp.float32),
    mesh=_mesh,
    scratch_types=[
        pltpu.VMEM((CH * CS,), jnp.int32),
        pltpu.VMEM((ACC_ROWS,), jnp.float32),
    ],
    compiler_params=_sc_params,
    name="sc_hist")


def _pair_body(h_h, pidx_h, out_h, pidx_v, rows_v, sem):
    c = lax.axis_index("c")
    s = lax.axis_index("s")
    wid = s * NC + c
    pltpu.sync_copy(pidx_h.at[wid], pidx_v)
    base = wid * (P // NW)
    for j in range(8):
        pltpu.async_copy(h_h.at[pidx_v.at[j]], rows_v, sem).wait()
        pltpu.sync_copy(
            rows_v, out_h.at[j // 4, pl.ds(base + (j % 4) * 128, 128)])


_pair_gather = pl.kernel(
    _pair_body,
    out_type=jax.ShapeDtypeStruct((2, P, D), jnp.float32),
    mesh=_mesh,
    scratch_types=[
        pltpu.VMEM((8, 128), jnp.int32),
        pltpu.VMEM((128, D), jnp.float32),
        pltpu.SemaphoreType.DMA,
    ],
    compiler_params=_sc_params,
    name="sc_pair_gather")


def _cnt_body(cnt_ref, inv_ref):
    tot = jnp.sum(cnt_ref[...], axis=0)
    inv_ref[...] = 1.0 / jnp.maximum(tot, 1.0)


def _cnt_reduce(cnts):
    return pl.pallas_call(
        _cnt_body,
        out_shape=jax.ShapeDtypeStruct((HR, 128), jnp.float32),
    )(cnts)


def _layer_tc_body(x_ref, p_ref, inv_ref, wr_ref, wn_ref, b_ref, o_ref, *, relu):
    agg = (p_ref[0] + p_ref[1]) * inv_ref[...]
    out = (jnp.dot(x_ref[...], wr_ref[...], preferred_element_type=jnp.float32)
           + jnp.dot(agg, wn_ref[...], preferred_element_type=jnp.float32)
           + b_ref[...])
    o_ref[...] = jnp.maximum(out, 0.0) if relu else out


def _layer_tc(x, parts, inv, wr, wn, b, relu):
    R = 2000
    grid = (N // R,)
    return pl.pallas_call(
        functools.partial(_layer_tc_body, relu=relu),
        grid=grid,
        in_specs=[
            pl.BlockSpec((R, D), lambda i: (i, 0)),
            pl.BlockSpec((NC, R, D), lambda i: (0, i, 0)),
            pl.BlockSpec((R, 1), lambda i: (i, 0)),
            pl.BlockSpec((D, D), lambda i: (0, 0)),
            pl.BlockSpec((D, D), lambda i: (0, 0)),
            pl.BlockSpec((1, D), lambda i: (0, 0)),
        ],
        out_specs=pl.BlockSpec((R, D), lambda i: (i, 0)),
        out_shape=jax.ShapeDtypeStruct((N, D), jnp.float32),
    )(x, parts, inv, wr, wn, b.reshape(1, D))


def _pred_body(hs_ref, hd_ref, w0_ref, b0_ref, w1_ref, b1_ref, w2_ref, b2_ref, o_ref):
    z = hs_ref[...] * hd_ref[...]
    z = jnp.maximum(jnp.dot(z, w0_ref[...], preferred_element_type=jnp.float32)
                    + b0_ref[...], 0.0)
    z = jnp.maximum(jnp.dot(z, w1_ref[...], preferred_element_type=jnp.float32)
                    + b1_ref[...], 0.0)
    logit = jnp.sum(z * w2_ref[...], axis=1, keepdims=True) + b2_ref[0, 0]
    o_ref[...] = 1.0 / (1.0 + jnp.exp(-logit))


def _predictor_tc(hs, hd, p0w, p0b, p1w, p1b, p2w, p2b):
    R = 2048
    grid = (P // R,)
    return pl.pallas_call(
        _pred_body,
        grid=grid,
        in_specs=[
            pl.BlockSpec((R, D), lambda i: (i, 0)),
            pl.BlockSpec((R, D), lambda i: (i, 0)),
            pl.BlockSpec((D, D), lambda i: (0, 0)),
            pl.BlockSpec((1, D), lambda i: (0, 0)),
            pl.BlockSpec((D, D), lambda i: (0, 0)),
            pl.BlockSpec((1, D), lambda i: (0, 0)),
            pl.BlockSpec((1, D), lambda i: (0, 0)),
            pl.BlockSpec((1, 1), lambda i: (0, 0)),
        ],
        out_specs=pl.BlockSpec((R, 1), lambda i: (i, 0)),
        out_shape=jax.ShapeDtypeStruct((P, 1), jnp.float32),
    )(hs, hd, p0w, p0b.reshape(1, D), p1w, p1b.reshape(1, D),
      p2w.reshape(1, D), p2b.reshape(1, 1))


def kernel(x, edge_index, pairs, W_root0, W_neigh0, b0, W_root1, W_neigh1, b1,
           P0_w, P0_b, P1_w, P1_b, P2_w, P2_b):
    dst = edge_index[1].reshape(NW, CH, CB)
    dst_p = jnp.pad(dst, ((0, 0), (0, 0), (0, CS - CB)), constant_values=DUMMY)

    pk_flat = jnp.bitwise_or(
        edge_index[0], jnp.left_shift(edge_index[1], 14)).reshape(NW, EW)
    packed = jnp.pad(pk_flat, ((0, 0), (0, EPW - EW)),
                     constant_values=PAD_PK).reshape(NW, PCH, PCB)

    cnts = _hist(dst_p.reshape(NW, CH * CS))
    inv = _cnt_reduce(cnts.reshape(NW, HR, 128)).reshape(ACC_ROWS, 1)

    parts0 = _segsum(x, packed)
    h = _layer_tc(x, parts0, inv, W_root0, W_neigh0, b0, relu=True)

    parts1 = _segsum(h, packed)
    h1 = _layer_tc(h, parts1, inv, W_root1, W_neigh1, b1, relu=False)

    pidx = pairs.reshape(2, NW, 4, 128).transpose(1, 0, 2, 3).reshape(NW, 8, 128)
    hp = _pair_gather(h1, pidx)

    return _predictor_tc(hp[0], hp[1], P0_w, P0_b, P1_w, P1_b, P2_w, P2_b)


# speedup vs baseline: 1.4161x; 1.4161x over previous
import functools

import jax
import jax.numpy as jnp
from jax import lax
from jax.experimental import pallas as pl
from jax.experimental.pallas import tpu as pltpu
from jax.experimental.pallas import tpu_sc as plsc

N = 10000
E = 320000
D = 128
P = 16384

NC = 2
NS = 16
NW = NC * NS
L = 16

EW = E // NW
CB = 125
CS = 128
CH = EW // CB
ACC_ROWS = 10240
HR = ACC_ROWS // 128
DUMMY = N
NB = 2

_mesh = plsc.VectorSubcoreMesh(
    core_axis_name="c", subcore_axis_name="s", num_cores=NC, num_subcores=NS)
_sc_params = pltpu.CompilerParams(needs_layout_passes=False)


def _seg_body(table_h, pidx_h, part_h,
              pidx_v, r0, r1, si0, si1, di0, di1, acc_s, g0, g1, s0, s1):
    c = lax.axis_index("c")
    s = lax.axis_index("s")
    wid = s * NC + c
    rows = (r0, r1)
    sib = (si0, si1)
    dib = (di0, di1)
    gsem = (g0, g1)
    ssem = (s0, s1)
    zeros16 = jnp.zeros((L,), jnp.float32)

    pltpu.sync_copy(pidx_h.at[wid], pidx_v)

    def unpack(j, b):
        for k in range(8):
            pk = pidx_v[j, pl.ds(k * L, L)]
            sib[b][pl.ds(k * L, L)] = lax.bitwise_and(pk, 0x3FFF)
            dib[b][pl.ds(k * L, L)] = lax.shift_right_logical(pk, 14)

    def zrow(i, _):
        for k in range(8):
            r0[i, pl.ds(k * L, L)] = zeros16
        return 0
    lax.fori_loop(0, CS, zrow, 0)
    for j in range(5):
        pltpu.sync_copy(r0, acc_s.at[pl.ds(s * 640 + j * CS, CS)])
    plsc.subcore_barrier()

    for b in range(NB):
        unpack(jnp.int32(b), b)
        pltpu.async_copy(table_h.at[sib[b]], rows[b], gsem[b])

    def group(g, _):
        base = g * NB
        for b in range(NB):
            pltpu.make_async_copy(table_h.at[sib[b]], rows[b], gsem[b]).wait()
            pltpu.async_copy(rows[b], acc_s.at[dib[b]], ssem[b], add=True)
        for b in range(NB):
            j = base + b
            pltpu.make_async_copy(rows[b], acc_s.at[dib[b]], ssem[b]).wait()
            jn = j + NB

            @pl.when(jn < CH)
            def _():
                unpack(jn, b)
                pltpu.async_copy(table_h.at[sib[b]], rows[b], gsem[b])
        return 0
    lax.fori_loop(0, CH // NB, group, 0)
    plsc.subcore_barrier()

    pltpu.sync_copy(acc_s.at[pl.ds(s * 640, 640)], part_h.at[c, pl.ds(s * 640, 640)])


_segsum = pl.kernel(
    _seg_body,
    out_type=jax.ShapeDtypeStruct((NC, ACC_ROWS, D), jnp.float32),
    mesh=_mesh,
    scratch_types=[
        pltpu.VMEM((CH, CS), jnp.int32),
        pltpu.VMEM((CS, D), jnp.float32),
        pltpu.VMEM((CS, D), jnp.float32),
        pltpu.VMEM((CS,), jnp.int32),
        pltpu.VMEM((CS,), jnp.int32),
        pltpu.VMEM((CS,), jnp.int32),
        pltpu.VMEM((CS,), jnp.int32),
        pltpu.VMEM_SHARED((ACC_ROWS, D), jnp.float32),
        pltpu.SemaphoreType.DMA,
        pltpu.SemaphoreType.DMA,
        pltpu.SemaphoreType.DMA,
        pltpu.SemaphoreType.DMA,
    ],
    compiler_params=_sc_params,
    name="sc_segsum")


def _hist_body(didx_h, cnt_h, didx_v, hist_v):
    c = lax.axis_index("c")
    s = lax.axis_index("s")
    wid = s * NC + c
    zeros16 = jnp.zeros((L,), jnp.float32)
    ones16 = jnp.ones((L,), jnp.float32)

    pltpu.sync_copy(didx_h.at[wid], didx_v)

    def zh(i, _):
        hist_v[pl.ds(i * L, L)] = zeros16
        return 0
    lax.fori_loop(0, ACC_ROWS // L, zh, 0)

    def step(i, _):
        di = didx_v[pl.ds(i * L, L)]
        plsc.addupdate_scatter(hist_v, [di], ones16)
        return 0
    lax.fori_loop(0, (CH * CS) // L, step, 0)

    pltpu.sync_copy(hist_v, cnt_h.at[pl.ds(wid * ACC_ROWS, ACC_ROWS)])


_hist = pl.kernel(
    _hist_body,
    out_type=jax.ShapeDtypeStruct((NW * ACC_ROWS,), jnp.float32),
    mesh=_mesh,
    scratch_types=[
        pltpu.VMEM((CH * CS,), jnp.int32),
        pltpu.VMEM((ACC_ROWS,), jnp.float32),
    ],
    compiler_params=_sc_params,
    name="sc_hist")


def _pair_body(h_h, pidx_h, out_h, pidx_v, rows_v, sem):
    c = lax.axis_index("c")
    s = lax.axis_index("s")
    wid = s * NC + c
    pltpu.sync_copy(pidx_h.at[wid], pidx_v)
    base = wid * (P // NW)
    for j in range(8):
        pltpu.async_copy(h_h.at[pidx_v.at[j]], rows_v, sem).wait()
        pltpu.sync_copy(
            rows_v, out_h.at[j // 4, pl.ds(base + (j % 4) * 128, 128)])


_pair_gather = pl.kernel(
    _pair_body,
    out_type=jax.ShapeDtypeStruct((2, P, D), jnp.float32),
    mesh=_mesh,
    scratch_types=[
        pltpu.VMEM((8, 128), jnp.int32),
        pltpu.VMEM((128, D), jnp.float32),
        pltpu.SemaphoreType.DMA,
    ],
    compiler_params=_sc_params,
    name="sc_pair_gather")


def _cnt_body(cnt_ref, inv_ref):
    tot = jnp.sum(cnt_ref[...], axis=0)
    inv_ref[...] = 1.0 / jnp.maximum(tot, 1.0)


def _cnt_reduce(cnts):
    return pl.pallas_call(
        _cnt_body,
        out_shape=jax.ShapeDtypeStruct((HR, 128), jnp.float32),
    )(cnts)


def _layer_tc_body(x_ref, p_ref, inv_ref, wr_ref, wn_ref, b_ref, o_ref, *, relu):
    agg = (p_ref[0] + p_ref[1]) * inv_ref[...]
    out = (jnp.dot(x_ref[...], wr_ref[...], preferred_element_type=jnp.float32)
           + jnp.dot(agg, wn_ref[...], preferred_element_type=jnp.float32)
           + b_ref[...])
    o_ref[...] = jnp.maximum(out, 0.0) if relu else out


def _layer_tc(x, parts, inv, wr, wn, b, relu):
    R = 2000
    grid = (N // R,)
    return pl.pallas_call(
        functools.partial(_layer_tc_body, relu=relu),
        grid=grid,
        in_specs=[
            pl.BlockSpec((R, D), lambda i: (i, 0)),
            pl.BlockSpec((NC, R, D), lambda i: (0, i, 0)),
            pl.BlockSpec((R, 1), lambda i: (i, 0)),
            pl.BlockSpec((D, D), lambda i: (0, 0)),
            pl.BlockSpec((D, D), lambda i: (0, 0)),
            pl.BlockSpec((1, D), lambda i: (0, 0)),
        ],
        out_specs=pl.BlockSpec((R, D), lambda i: (i, 0)),
        out_shape=jax.ShapeDtypeStruct((N, D), jnp.float32),
    )(x, parts, inv, wr, wn, b.reshape(1, D))


def _pred_body(hs_ref, hd_ref, w0_ref, b0_ref, w1_ref, b1_ref, w2_ref, b2_ref, o_ref):
    z = hs_ref[...] * hd_ref[...]
    z = jnp.maximum(jnp.dot(z, w0_ref[...], preferred_element_type=jnp.float32)
                    + b0_ref[...], 0.0)
    z = jnp.maximum(jnp.dot(z, w1_ref[...], preferred_element_type=jnp.float32)
                    + b1_ref[...], 0.0)
    logit = jnp.sum(z * w2_ref[...], axis=1, keepdims=True) + b2_ref[0, 0]
    o_ref[...] = 1.0 / (1.0 + jnp.exp(-logit))


def _predictor_tc(hs, hd, p0w, p0b, p1w, p1b, p2w, p2b):
    R = 2048
    grid = (P // R,)
    return pl.pallas_call(
        _pred_body,
        grid=grid,
        in_specs=[
            pl.BlockSpec((R, D), lambda i: (i, 0)),
            pl.BlockSpec((R, D), lambda i: (i, 0)),
            pl.BlockSpec((D, D), lambda i: (0, 0)),
            pl.BlockSpec((1, D), lambda i: (0, 0)),
            pl.BlockSpec((D, D), lambda i: (0, 0)),
            pl.BlockSpec((1, D), lambda i: (0, 0)),
            pl.BlockSpec((1, D), lambda i: (0, 0)),
            pl.BlockSpec((1, 1), lambda i: (0, 0)),
        ],
        out_specs=pl.BlockSpec((R, 1), lambda i: (i, 0)),
        out_shape=jax.ShapeDtypeStruct((P, 1), jnp.float32),
    )(hs, hd, p0w, p0b.reshape(1, D), p1w, p1b.reshape(1, D),
      p2w.reshape(1, D), p2b.reshape(1, 1))


def kernel(x, edge_index, pairs, W_root0, W_neigh0, b0, W_root1, W_neigh1, b1,
           P0_w, P0_b, P1_w, P1_b, P2_w, P2_b):
    src = edge_index[0].reshape(NW, CH, CB)
    dst = edge_index[1].reshape(NW, CH, CB)
    src_p = jnp.pad(src, ((0, 0), (0, 0), (0, CS - CB)))
    dst_p = jnp.pad(dst, ((0, 0), (0, 0), (0, CS - CB)), constant_values=DUMMY)
    packed = jnp.bitwise_or(src_p, jnp.left_shift(dst_p, 14))

    cnts = _hist(dst_p.reshape(NW, CH * CS))
    inv = _cnt_reduce(cnts.reshape(NW, HR, 128)).reshape(ACC_ROWS, 1)

    parts0 = _segsum(x, packed)
    h = _layer_tc(x, parts0, inv, W_root0, W_neigh0, b0, relu=True)

    parts1 = _segsum(h, packed)
    h1 = _layer_tc(h, parts1, inv, W_root1, W_neigh1, b1, relu=False)

    pidx = pairs.reshape(2, NW, 4, 128).transpose(1, 0, 2, 3).reshape(NW, 8, 128)
    hp = _pair_gather(h1, pidx)

    return _predictor_tc(hp[0], hp[1], P0_w, P0_b, P1_w, P1_b, P2_w, P2_b)
